# baseline (speedup 1.0000x reference)
import functools

import jax
import jax.numpy as jnp
from jax import lax
from jax.experimental import pallas as pl
from jax.experimental.pallas import tpu as pltpu
from jax.experimental.pallas import tpu_sc as plsc

NC = 2
NS = 16
L = 16
G = 16
SRC_BITS = 14
GB = 2048


def _sc_aggregate(hc_lo, hc_hi, srcp, dstp, npw, ch):
    n_pad = npw * NC * NS
    per_sc = NS * npw
    hdim = hc_lo.shape[1]
    half = hdim // 2
    e_pad = srcp.shape[0]
    nch = e_pad // ch
    nvr = ch // L
    ncol = hdim // L
    nmaxcol = half // L
    el = e_pad + L * nch + ch + L
    mesh = plsc.VectorSubcoreMesh(core_axis_name="c", subcore_axis_name="s")

    @functools.partial(
        pl.kernel,
        mesh=mesh,
        compiler_params=pltpu.CompilerParams(needs_layout_passes=False),
        out_type=[
            jax.ShapeDtypeStruct((n_pad, hdim), jnp.float32),
            jax.ShapeDtypeStruct((n_pad, hdim), jnp.float32),
            jax.ShapeDtypeStruct((n_pad, half), jnp.float32),
            jax.ShapeDtypeStruct((n_pad, half), jnp.float32),
            jax.ShapeDtypeStruct((NC * NS * el,), jnp.int32),
        ],
        scratch_types=[
            pltpu.VMEM_SHARED((per_sc, hdim), jnp.float32),
            pltpu.VMEM((npw, half), jnp.float32),
            pltpu.VMEM((ch,), jnp.int32),
            pltpu.VMEM((ch,), jnp.int32),
            pltpu.VMEM((ch + L,), jnp.int32),
            pltpu.VMEM((GB,), jnp.int32),
            pltpu.VMEM((G, hdim), jnp.float32),
        ],
    )
    def sc_k(hc_lo_hbm, hc_hi_hbm, src_hbm, dst_hbm,
             sum_lo_out, sum_hi_out, max_lo_out, max_hi_out, elist,
             sum_sh, acc_max, dst_v, src_v, sel, lstbuf, buf):
        cid = lax.axis_index("c")
        sid = lax.axis_index("s")
        wid = cid * NS + sid
        base_g = cid * per_sc + sid * npw
        base_sc = sid * npw

        zero16 = jnp.zeros((L,), jnp.float32)
        ninf16 = jnp.full((L,), -jnp.inf, jnp.float32)
        isent16 = jnp.full((L,), -1, jnp.int32)

        def zero_buf(t, carry):
            for col in range(ncol):
                buf[t, pl.ds(col * L, L)] = zero16
            return carry

        lanes = lax.iota(jnp.int32, L)

        def chunk_body(k, tot_g):
            pltpu.sync_copy(dst_hbm.at[pl.ds(k * ch, ch)], dst_v)
            pltpu.sync_copy(src_hbm.at[pl.ds(k * ch, ch)], src_v)

            def scan_body(j, cnt):
                d = dst_v[pl.ds(j * L, L)]
                sv = src_v[pl.ds(j * L, L)]
                m = (d >= base_g) & (d < base_g + npw)
                incl = plsc.cumsum(m.astype(jnp.int32))
                pos = cnt + incl - 1
                packed = ((d - base_g) << SRC_BITS) | sv
                plsc.store_scatter(sel, [pos], packed, mask=m)
                return cnt + incl[L - 1]

            cnt = lax.fori_loop(0, nvr, scan_body, 0)
            plsc.store_scatter(sel, [cnt + lanes], isent16)
            pltpu.sync_copy(
                sel, elist.at[pl.ds((ebase_g + tot_g) * L, ch + L)])
            return tot_g + (cnt + L - 1) // L

        ebase_g = wid * (el // L)
        total_g = lax.fori_loop(0, nch, chunk_body, 0)

        def sent_fill(j, carry):
            sel[pl.ds(j * L, L)] = isent16
            return carry

        lax.fori_loop(0, (ch + L) // L, sent_fill, 0)
        pltpu.sync_copy(sel, elist.at[pl.ds((ebase_g + total_g) * L, ch + L)])

        nblk = (total_g * L + GB - 1) // GB

        for hv in range(2):
            hc_hbm = hc_lo_hbm if hv == 0 else hc_hi_hbm
            sum_out = sum_lo_out if hv == 0 else sum_hi_out
            max_out = max_lo_out if hv == 0 else max_hi_out

            def init_max(r, carry):
                for col in range(nmaxcol):
                    acc_max[r, pl.ds(col * L, L)] = ninf16
                return carry

            lax.fori_loop(0, npw, init_max, 0)
            lax.fori_loop(0, G, zero_buf, 0)

            def zero_sh(r, carry):
                pltpu.sync_copy(buf, sum_sh.at[pl.ds(base_sc + r * G, G)])
                return carry

            lax.fori_loop(0, npw // G, zero_sh, 0)

            def blk_body(b, carry):
                pltpu.sync_copy(
                    elist.at[pl.ds((ebase_g + b * (GB // L)) * L, GB)],
                    lstbuf)

                def group_body(g, icarry):
                    pv = lstbuf[pl.ds(g * L, L)]
                    valid = pv >= 0
                    sidx = jnp.where(valid, pv & ((1 << SRC_BITS) - 1), 0)
                    tl_vec = pv >> SRC_BITS
                    didx = jnp.where(valid, tl_vec + base_sc, base_sc)
                    pltpu.sync_copy(hc_hbm.at[sidx], buf)
                    for t in range(G):
                        @pl.when(pv[t] < 0)
                        def _(t=t):
                            for col in range(ncol):
                                buf[t, pl.ds(col * L, L)] = zero16
                    pltpu.sync_copy(buf, sum_sh.at[didx], add=True)
                    for t in range(G):
                        @pl.when(pv[t] >= 0)
                        def _(t=t):
                            tl = tl_vec[t]
                            for col in range(nmaxcol):
                                v = buf[t, pl.ds(col * L, L)]
                                a = acc_max[tl, pl.ds(col * L, L)]
                                acc_max[tl, pl.ds(col * L, L)] = (
                                    jnp.maximum(a, v))
                    return icarry

                lax.fori_loop(0, GB // L, group_body, 0)
                return carry

            lax.fori_loop(0, nblk, blk_body, 0)

            pltpu.sync_copy(acc_max, max_out.at[pl.ds(base_g, npw)])
            pltpu.sync_copy(sum_sh.at[pl.ds(base_sc, npw)],
                            sum_out.at[pl.ds(base_g, npw)])

    return sc_k(hc_lo, hc_hi, srcp, dstp)


def _tc_body(x_ref, c_ref, slo_ref, shi_ref, mlo_ref, mhi_ref,
             wt_ref, ut_ref, b_ref, uft_ref, ufb_ref, hn_ref, cn_ref,
             *, hdim):
    half = hdim // 2
    slo = slo_ref[...]
    shi = shi_ref[...]
    hs = jnp.concatenate([slo[:, :half], shi[:, :half]], axis=1)
    cs = jnp.concatenate([slo[:, half:], shi[:, half:]], axis=1)
    hmr = jnp.concatenate([mlo_ref[...], mhi_ref[...]], axis=1)
    has = hmr[:, 0:1] != -jnp.inf
    hm = jnp.where(hmr == -jnp.inf, 0.0, hmr)
    hcomb = jnp.concatenate([hs, hm], axis=1)
    f = jax.nn.sigmoid(
        jnp.dot(hcomb, uft_ref[...], preferred_element_type=jnp.float32)
        + ufb_ref[...])
    iou_red = jnp.dot(hcomb, ut_ref[...], preferred_element_type=jnp.float32)
    iou_leaf = jnp.dot(x_ref[...], wt_ref[...],
                       preferred_element_type=jnp.float32)
    iou = jnp.where(has, iou_red, iou_leaf) + b_ref[...]
    c_data = jnp.where(has, f * cs, c_ref[...])
    i = jax.nn.sigmoid(iou[:, :hdim])
    o = jax.nn.sigmoid(iou[:, hdim:2 * hdim])
    u = jnp.tanh(iou[:, 2 * hdim:])
    c_new = i * u + c_data
    cn_ref[...] = c_new
    hn_ref[...] = o * jnp.tanh(c_new)


def _tc_apply(xp, cp, slo, shi, mlo, mhi, wt, ut, b, uft, ufb):
    n_pad, xdim = xp.shape
    hdim = cp.shape[1]
    half = hdim // 2
    br = 256
    grid = (n_pad // br,)
    row = lambda i: (i, 0)
    fixed = lambda i: (0, 0)
    return pl.pallas_call(
        functools.partial(_tc_body, hdim=hdim),
        grid=grid,
        in_specs=[
            pl.BlockSpec((br, xdim), row),
            pl.BlockSpec((br, hdim), row),
            pl.BlockSpec((br, hdim), row),
            pl.BlockSpec((br, hdim), row),
            pl.BlockSpec((br, half), row),
            pl.BlockSpec((br, half), row),
            pl.BlockSpec((xdim, 3 * hdim), fixed),
            pl.BlockSpec((2 * hdim, 3 * hdim), fixed),
            pl.BlockSpec((1, 3 * hdim), fixed),
            pl.BlockSpec((2 * hdim, hdim), fixed),
            pl.BlockSpec((1, hdim), fixed),
        ],
        out_specs=[
            pl.BlockSpec((br, hdim), row),
            pl.BlockSpec((br, hdim), row),
        ],
        out_shape=[
            jax.ShapeDtypeStruct((n_pad, hdim), jnp.float32),
            jax.ShapeDtypeStruct((n_pad, hdim), jnp.float32),
        ],
    )(xp, cp, slo, shi, mlo, mhi, wt, ut, b, uft, ufb)


def kernel(x, h, c, edge_index, W_iou, U_iou, b_iou, U_f_w, U_f_b):
    n, xdim = x.shape
    hdim = h.shape[1]
    half = hdim // 2
    e = edge_index.shape[1]

    nw = NC * NS
    npw = -(-n // nw)
    npw = -(-npw // L) * L
    n_pad = npw * nw

    ch = 8000
    e_pad = -(-e // ch) * ch
    src = edge_index[0]
    dst = edge_index[1]
    if e_pad != e:
        src = jnp.concatenate([src, jnp.zeros((e_pad - e,), jnp.int32)])
        dst = jnp.concatenate(
            [dst, jnp.full((e_pad - e,), n_pad, jnp.int32)])

    hc_lo = jnp.concatenate([h[:, :half], c[:, :half]], axis=1)
    hc_hi = jnp.concatenate([h[:, half:], c[:, half:]], axis=1)

    slo, shi, mlo, mhi, _ = _sc_aggregate(hc_lo, hc_hi, src, dst, npw, ch)

    pad_n = ((0, n_pad - n), (0, 0))
    xp = jnp.pad(x, pad_n)
    cp = jnp.pad(c, pad_n)
    hn, cn = _tc_apply(xp, cp, slo, shi, mlo, mhi,
                       W_iou.T, U_iou.T, b_iou, U_f_w.T,
                       U_f_b.reshape(1, hdim))
    return hn[:n], cn[:n]
